# baseline (speedup 1.0000x reference)
import functools

import jax
import jax.numpy as jnp
from jax import lax
from jax.experimental import pallas as pl
from jax.experimental.pallas import tpu as pltpu
from jax.experimental.pallas import tpu_sc as plsc

_NUM_CODES = 1024
_ROWS = 18432
_BLK = 512
_GRID = _ROWS // _BLK

_COLOR_D = 2250
_POS_D = 450
_COLOR_DP = 2304
_POS_DP = 512
_COLOR_MAIN = 2176
_POS_MAIN = 384
_COLOR_TAIL = _COLOR_D - _COLOR_MAIN
_POS_TAIL = _POS_D - _POS_MAIN

_NC = 2
_NS = 16
_NW = _NC * _NS
_BPW = _ROWS // _NW
_CH = 16
_NCHUNK = _BPW // _CH


def _stats_body(x_ref, idx_ref, ent_ref, acc_ref):
    i = pl.program_id(0)
    x = x_ref[...]
    m = jnp.max(x, axis=-1, keepdims=True)
    ids = lax.broadcasted_iota(jnp.int32, x.shape, 1)
    idx = jnp.min(jnp.where(x == m, ids, _NUM_CODES), axis=-1)
    idx_ref[...] = idx[None, None, :]

    e = jnp.exp(x - m)
    s = jnp.sum(e, axis=-1, keepdims=True)
    p = e / s

    @pl.when(i == 0)
    def _():
        acc_ref[...] = jnp.zeros_like(acc_ref)

    acc_ref[...] += jnp.sum(p, axis=0, keepdims=True)

    @pl.when(i == pl.num_programs(0) - 1)
    def _():
        avg = acc_ref[...] * (1.0 / _ROWS)
        ent_ref[0, 0] = -jnp.sum(avg * jnp.log(avg + 1e-8))


_stats_call = pl.pallas_call(
    _stats_body,
    grid=(_GRID,),
    in_specs=[pl.BlockSpec((_BLK, _NUM_CODES), lambda i: (i, 0))],
    out_specs=[
        pl.BlockSpec((1, 1, _BLK), lambda i: (i, 0, 0)),
        pl.BlockSpec(block_shape=(1, 1), index_map=lambda i: (0, 0),
                     memory_space=pltpu.SMEM),
    ],
    out_shape=[
        jax.ShapeDtypeStruct((_GRID, 1, _BLK), jnp.int32),
        jax.ShapeDtypeStruct((1, 1), jnp.float32),
    ],
    scratch_shapes=[pltpu.VMEM((1, _NUM_CODES), jnp.float32)],
)


def _compact_tail(src_v, slot, dst, col0, ncol):
    nfull = ncol // 16
    rem = ncol - nfull * 16
    lanes = lax.iota(jnp.int32, 16)
    mask = lanes < rem
    for r in range(16):
        for wv in range(nfull):
            dst[r, pl.ds(16 * wv, 16)] = src_v[slot, r, pl.ds(col0 + 16 * wv, 16)]
        v = src_v[slot, r, pl.ds(col0 + 16 * nfull, 16)]
        plsc.store_scatter(dst, [jnp.full((16,), r, jnp.int32),
                                 16 * nfull + lanes], v, mask=mask)


def _gather_body(idx_hbm, color_hbm, pos_hbm, outc_hbm, outp_hbm,
                 idx_v, c_v, p_v, ct_v, pt_v, sem0, sem1):
    wid = lax.axis_index("s") * _NC + lax.axis_index("c")
    row0 = wid * _BPW
    sems = (sem0, sem1)

    def start_chunk(c, slot):
        pltpu.sync_copy(idx_hbm.at[pl.ds(row0 + c * _CH, _CH)],
                        idx_v.at[slot])
        pltpu.async_copy(color_hbm.at[idx_v.at[slot]], c_v.at[slot],
                         sems[slot])
        pltpu.async_copy(pos_hbm.at[idx_v.at[slot]], p_v.at[slot],
                         sems[slot])

    def wait_chunk(slot):
        pltpu.make_async_copy(color_hbm.at[idx_v.at[slot]], c_v.at[slot],
                              sems[slot]).wait()
        pltpu.make_async_copy(pos_hbm.at[idx_v.at[slot]], p_v.at[slot],
                              sems[slot]).wait()

    def writeback(c, slot):
        base = row0 + c * _CH
        _compact_tail(c_v, slot, ct_v, _COLOR_MAIN, _COLOR_TAIL)
        _compact_tail(p_v, slot, pt_v, _POS_MAIN, _POS_TAIL)
        pltpu.sync_copy(c_v.at[slot, :, pl.ds(0, _COLOR_MAIN)],
                        outc_hbm.at[pl.ds(base, _CH), pl.ds(0, _COLOR_MAIN)])
        pltpu.sync_copy(ct_v,
                        outc_hbm.at[pl.ds(base, _CH),
                                    pl.ds(_COLOR_MAIN, _COLOR_TAIL)])
        pltpu.sync_copy(p_v.at[slot, :, pl.ds(0, _POS_MAIN)],
                        outp_hbm.at[pl.ds(base, _CH), pl.ds(0, _POS_MAIN)])
        pltpu.sync_copy(pt_v,
                        outp_hbm.at[pl.ds(base, _CH),
                                    pl.ds(_POS_MAIN, _POS_TAIL)])

    start_chunk(0, 0)

    def pair(g, carry):
        c0 = 2 * g
        wait_chunk(0)
        start_chunk(c0 + 1, 1)
        writeback(c0, 0)
        wait_chunk(1)

        @pl.when(c0 + 2 < _NCHUNK)
        def _():
            start_chunk(c0 + 2, 0)

        writeback(c0 + 1, 1)
        return carry

    lax.fori_loop(0, _NCHUNK // 2, pair, 0)


@functools.lru_cache(maxsize=1)
def _make_gather_call():
    return pl.kernel(
        _gather_body,
        out_type=[
            jax.ShapeDtypeStruct((_ROWS, _COLOR_D), jnp.float32),
            jax.ShapeDtypeStruct((_ROWS, _POS_D), jnp.float32),
        ],
        mesh=plsc.VectorSubcoreMesh(core_axis_name="c", subcore_axis_name="s"),
        scratch_types=[
            pltpu.VMEM((2, _CH), jnp.int32),
            pltpu.VMEM((2, _CH, _COLOR_DP), jnp.float32),
            pltpu.VMEM((2, _CH, _POS_DP), jnp.float32),
            pltpu.VMEM((_CH, _COLOR_TAIL), jnp.float32),
            pltpu.VMEM((_CH, _POS_TAIL), jnp.float32),
            pltpu.SemaphoreType.DMA,
            pltpu.SemaphoreType.DMA,
        ],
        compiler_params=pltpu.CompilerParams(needs_layout_passes=False),
    )


_pos_fix = pl.pallas_call(
    lambda x_ref, o_ref: o_ref.__setitem__((...,), x_ref[...]),
    grid=(_GRID,),
    in_specs=[pl.BlockSpec((_BLK, _POS_D), lambda i: (i, 0))],
    out_specs=pl.BlockSpec((_BLK, _POS_D), lambda i: (i, 0)),
    out_shape=jax.ShapeDtypeStruct((_ROWS, _POS_D), jnp.float32),
)


def kernel(logits, color_basis, pos_basis):
    b, k, c = logits.shape
    lf = logits.reshape(b * k, c)
    idx3d, ent = _stats_call(lf)
    idx_flat = idx3d.reshape(-1)
    color_p = jnp.pad(color_basis, ((0, 0), (0, _COLOR_DP - _COLOR_D)))
    pos_p = jnp.pad(pos_basis, ((0, 0), (0, _POS_DP - _POS_D)))
    colm, posm = _make_gather_call()(idx_flat, color_p, pos_p)
    posm = _pos_fix(posm)
    return (
        colm.reshape(b, k, _COLOR_D),
        posm.reshape(b, k, _POS_D),
        idx_flat.reshape(b, k),
        ent[0, 0],
    )


# speedup vs baseline: 1.4972x; 1.4972x over previous
import functools

import jax
import jax.numpy as jnp
from jax import lax
from jax.experimental import pallas as pl
from jax.experimental.pallas import tpu as pltpu
from jax.experimental.pallas import tpu_sc as plsc

_NUM_CODES = 1024
_ROWS = 18432
_BLK = 512
_GRID = _ROWS // _BLK

_COLOR_D = 2250
_POS_D = 450
_COLOR_DP = 2304
_POS_DP = 512
_COLOR_MAIN = 2176
_POS_MAIN = 384

_NC = 2
_NS = 16
_NW = _NC * _NS
_BPW = _ROWS // _NW


def _stats_body(x_ref, idx_ref, ent_ref, acc_ref):
    i = pl.program_id(0)
    x = x_ref[...]
    m = jnp.max(x, axis=-1, keepdims=True)
    ids = lax.broadcasted_iota(jnp.int32, x.shape, 1)
    idx = jnp.min(jnp.where(x == m, ids, _NUM_CODES), axis=-1)
    idx_ref[...] = idx[None, None, :]

    e = jnp.exp(x - m)
    s = jnp.sum(e, axis=-1, keepdims=True)
    p = e / s

    @pl.when(i == 0)
    def _():
        acc_ref[...] = jnp.zeros_like(acc_ref)

    acc_ref[...] += jnp.sum(p, axis=0, keepdims=True)

    @pl.when(i == pl.num_programs(0) - 1)
    def _():
        avg = acc_ref[...] * (1.0 / _ROWS)
        ent_ref[0, 0] = -jnp.sum(avg * jnp.log(avg + 1e-8))


_stats_call = pl.pallas_call(
    _stats_body,
    grid=(_GRID,),
    in_specs=[pl.BlockSpec((_BLK, _NUM_CODES), lambda i: (i, 0))],
    out_specs=[
        pl.BlockSpec((1, 1, _BLK), lambda i: (i, 0, 0)),
        pl.BlockSpec(block_shape=(1, 1), index_map=lambda i: (0, 0),
                     memory_space=pltpu.SMEM),
    ],
    out_shape=[
        jax.ShapeDtypeStruct((_GRID, 1, _BLK), jnp.int32),
        jax.ShapeDtypeStruct((1, 1), jnp.float32),
    ],
    scratch_shapes=[pltpu.VMEM((1, _NUM_CODES), jnp.float32)],
)


_pos_transpose = pl.pallas_call(
    lambda x_ref, o_ref: o_ref.__setitem__((...,), x_ref[...].T),
    grid=(_GRID,),
    in_specs=[pl.BlockSpec((_BLK, _POS_D), lambda i: (i, 0))],
    out_specs=pl.BlockSpec((_POS_D, _BLK), lambda i: (0, i)),
    out_shape=jax.ShapeDtypeStruct((_POS_D, _ROWS), jnp.float32),
)


def _compact_tail(src_v, slot, dst, col0, ncol, ch):
    nfull = ncol // 16
    rem = ncol - nfull * 16
    lanes = lax.iota(jnp.int32, 16)
    mask = lanes < rem
    for r in range(ch):
        for wv in range(nfull):
            dst[r, pl.ds(16 * wv, 16)] = src_v[slot, r,
                                               pl.ds(col0 + 16 * wv, 16)]
        v = src_v[slot, r, pl.ds(col0 + 16 * nfull, 16)]
        plsc.store_scatter(dst, [jnp.full((16,), r, jnp.int32),
                                 16 * nfull + lanes], v, mask=mask)


def _sc_gather_body(dp, d, main, ch, idx_hbm, tab_hbm, out_hbm,
                    idx_v, g_v, t_v, sem0, sem1):
    nchunk = _BPW // ch
    tail = d - main
    wid = lax.axis_index("s") * _NC + lax.axis_index("c")
    row0 = wid * _BPW
    sems = (sem0, sem1)

    def start_chunk(c, slot):
        pltpu.sync_copy(idx_hbm.at[pl.ds(row0 + c * ch, ch)], idx_v.at[slot])
        pltpu.async_copy(tab_hbm.at[idx_v.at[slot]], g_v.at[slot],
                         sems[slot])

    def wait_chunk(slot):
        pltpu.make_async_copy(tab_hbm.at[idx_v.at[slot]], g_v.at[slot],
                              sems[slot]).wait()

    def writeback(c, slot):
        base = row0 + c * ch
        _compact_tail(g_v, slot, t_v, main, tail, ch)
        pltpu.sync_copy(g_v.at[slot, :, pl.ds(0, main)],
                        out_hbm.at[pl.ds(base, ch), pl.ds(0, main)])
        pltpu.sync_copy(t_v, out_hbm.at[pl.ds(base, ch), pl.ds(main, tail)])

    start_chunk(0, 0)

    def pair(g, carry):
        c0 = 2 * g
        wait_chunk(0)
        start_chunk(c0 + 1, 1)
        writeback(c0, 0)
        wait_chunk(1)

        @pl.when(c0 + 2 < nchunk)
        def _():
            start_chunk(c0 + 2, 0)

        writeback(c0 + 1, 1)
        return carry

    lax.fori_loop(0, nchunk // 2, pair, 0)


@functools.lru_cache(maxsize=None)
def _make_sc_gather(dp, d, main, ch):
    tail = d - main
    return pl.kernel(
        functools.partial(_sc_gather_body, dp, d, main, ch),
        out_type=jax.ShapeDtypeStruct((_ROWS, d), jnp.float32),
        mesh=plsc.VectorSubcoreMesh(core_axis_name="c", subcore_axis_name="s"),
        scratch_types=[
            pltpu.VMEM((2, ch), jnp.int32),
            pltpu.VMEM((2, ch, dp), jnp.float32),
            pltpu.VMEM((ch, tail), jnp.float32),
            pltpu.SemaphoreType.DMA,
            pltpu.SemaphoreType.DMA,
        ],
        compiler_params=pltpu.CompilerParams(needs_layout_passes=False),
    )


def kernel(logits, color_basis, pos_basis):
    b, k, c = logits.shape
    lf = logits.reshape(b * k, c)
    idx3d, ent = _stats_call(lf)
    idx_flat = idx3d.reshape(-1)
    color_p = jnp.pad(color_basis, ((0, 0), (0, _COLOR_DP - _COLOR_D)))
    pos_p = jnp.pad(pos_basis, ((0, 0), (0, _POS_DP - _POS_D)))
    posm = _make_sc_gather(_POS_DP, _POS_D, _POS_MAIN, 32)(
        idx_flat, pos_p)
    colm = _make_sc_gather(_COLOR_DP, _COLOR_D, _COLOR_MAIN, 16)(
        idx_flat, color_p)
    pos_t = _pos_transpose(posm)
    posm_out = jnp.transpose(pos_t.reshape(_POS_D, b, k), (1, 2, 0))
    return (
        colm.reshape(b, k, _COLOR_D),
        posm_out,
        idx_flat.reshape(b, k),
        ent[0, 0],
    )


# speedup vs baseline: 1.5725x; 1.0503x over previous
import functools

import jax
import jax.numpy as jnp
from jax import lax
from jax.experimental import pallas as pl
from jax.experimental.pallas import tpu as pltpu
from jax.experimental.pallas import tpu_sc as plsc

_NUM_CODES = 1024
_ROWS = 18432
_BLK = 512
_GRID = _ROWS // _BLK

_COLOR_D = 2250
_POS_D = 450
_COLOR_DP = 2304
_POS_DP = 512
_COLOR_MAIN = 2176
_POS_MAIN = 384

_NC = 2
_NS = 16
_NW = _NC * _NS
_BPW = _ROWS // _NW


def _stats_body(x_ref, idx_ref, ent_ref, acc_ref):
    i = pl.program_id(0)
    x = x_ref[...]
    m = jnp.max(x, axis=-1, keepdims=True)
    ids = lax.broadcasted_iota(jnp.int32, x.shape, 1)
    idx = jnp.min(jnp.where(x == m, ids, _NUM_CODES), axis=-1)
    idx_ref[...] = idx[None, None, :]

    e = jnp.exp(x - m)
    s = jnp.sum(e, axis=-1, keepdims=True)
    p = e / s

    @pl.when(i == 0)
    def _():
        acc_ref[...] = jnp.zeros_like(acc_ref)

    acc_ref[...] += jnp.sum(p, axis=0, keepdims=True)

    @pl.when(i == pl.num_programs(0) - 1)
    def _():
        avg = acc_ref[...] * (1.0 / _ROWS)
        ent_ref[0, 0] = -jnp.sum(avg * jnp.log(avg + 1e-8))


_stats_call = pl.pallas_call(
    _stats_body,
    grid=(_GRID,),
    in_specs=[pl.BlockSpec((_BLK, _NUM_CODES), lambda i: (i, 0))],
    out_specs=[
        pl.BlockSpec((1, 1, _BLK), lambda i: (i, 0, 0)),
        pl.BlockSpec(block_shape=(1, 1), index_map=lambda i: (0, 0),
                     memory_space=pltpu.SMEM),
    ],
    out_shape=[
        jax.ShapeDtypeStruct((_GRID, 1, _BLK), jnp.int32),
        jax.ShapeDtypeStruct((1, 1), jnp.float32),
    ],
    scratch_shapes=[pltpu.VMEM((1, _NUM_CODES), jnp.float32)],
)


_pos_transpose = pl.pallas_call(
    lambda x_ref, o_ref: o_ref.__setitem__((...,), x_ref[...].T),
    grid=(_GRID,),
    in_specs=[pl.BlockSpec((_BLK, _POS_D), lambda i: (i, 0))],
    out_specs=pl.BlockSpec((_POS_D, _BLK), lambda i: (0, i)),
    out_shape=jax.ShapeDtypeStruct((_POS_D, _ROWS), jnp.float32),
)


def _compact_tail(src_v, slot, dst_v, col0, ncol, ch):
    nfull = ncol // 16
    rem = ncol - nfull * 16
    lanes = lax.iota(jnp.int32, 16)
    mask = lanes < rem
    for r in range(ch):
        for wv in range(nfull):
            dst_v[slot, r, pl.ds(16 * wv, 16)] = src_v[slot, r,
                                                       pl.ds(col0 + 16 * wv, 16)]
        v = src_v[slot, r, pl.ds(col0 + 16 * nfull, 16)]
        plsc.store_scatter(dst_v.at[slot],
                           [jnp.full((16,), r, jnp.int32), 16 * nfull + lanes],
                           v, mask=mask)


_NSLOT = 3


def _sc_gather_body(dp, d, main, ch, idx_hbm, tab_hbm, out_hbm,
                    idx_v, g_v, t_v, gsems, wsems):
    nchunk = _BPW // ch
    tail = d - main
    wid = lax.axis_index("s") * _NC + lax.axis_index("c")
    row0 = wid * _BPW

    pltpu.sync_copy(idx_hbm.at[pl.ds(row0, _BPW)], idx_v)

    def gather_pair(c, slot):
        return pltpu.make_async_copy(
            tab_hbm.at[idx_v.at[pl.ds(c * ch, ch)]], g_v.at[slot],
            gsems[slot])

    def wb_pairs(c, slot):
        base = row0 + c * ch
        return (
            pltpu.make_async_copy(
                g_v.at[slot, :, pl.ds(0, main)],
                out_hbm.at[pl.ds(base, ch), pl.ds(0, main)], wsems[slot]),
            pltpu.make_async_copy(
                t_v.at[slot], out_hbm.at[pl.ds(base, ch), pl.ds(main, tail)],
                wsems[slot]),
        )

    def start_wb(c, slot):
        _compact_tail(g_v, slot, t_v, main, tail, ch)
        for cp in wb_pairs(c, slot):
            cp.start()

    def wait_wb(c, slot):
        for cp in wb_pairs(c, slot):
            cp.wait()

    gather_pair(0, 0).start()
    gather_pair(1, 1).start()

    def triple(g, carry):
        c0 = _NSLOT * g
        for j in range(_NSLOT):
            c = c0 + j
            slot = j
            nslot = (j + 2) % _NSLOT
            gather_pair(c, slot).wait()

            @pl.when(c + 2 < nchunk)
            def _():
                @pl.when(c >= 1)
                def _():
                    wait_wb(c - 1, nslot)
                gather_pair(c + 2, nslot).start()

            start_wb(c, slot)
        return carry

    lax.fori_loop(0, nchunk // _NSLOT, triple, 0)
    for c in range(nchunk - _NSLOT, nchunk):
        wait_wb(c, c % _NSLOT)


@functools.lru_cache(maxsize=None)
def _make_sc_gather(dp, d, main, ch):
    tail = d - main
    return pl.kernel(
        functools.partial(_sc_gather_body, dp, d, main, ch),
        out_type=jax.ShapeDtypeStruct((_ROWS, d), jnp.float32),
        mesh=plsc.VectorSubcoreMesh(core_axis_name="c", subcore_axis_name="s"),
        scratch_types=[
            pltpu.VMEM((_BPW,), jnp.int32),
            pltpu.VMEM((_NSLOT, ch, dp), jnp.float32),
            pltpu.VMEM((_NSLOT, ch, tail), jnp.float32),
            [pltpu.SemaphoreType.DMA] * _NSLOT,
            [pltpu.SemaphoreType.DMA] * _NSLOT,
        ],
        compiler_params=pltpu.CompilerParams(needs_layout_passes=False),
    )


def kernel(logits, color_basis, pos_basis):
    b, k, c = logits.shape
    lf = logits.reshape(b * k, c)
    idx3d, ent = _stats_call(lf)
    idx_flat = idx3d.reshape(-1)
    color_p = jnp.pad(color_basis, ((0, 0), (0, _COLOR_DP - _COLOR_D)))
    pos_p = jnp.pad(pos_basis, ((0, 0), (0, _POS_DP - _POS_D)))
    posm = _make_sc_gather(_POS_DP, _POS_D, _POS_MAIN, 48)(
        idx_flat, pos_p)
    colm = _make_sc_gather(_COLOR_DP, _COLOR_D, _COLOR_MAIN, 16)(
        idx_flat, color_p)
    pos_t = _pos_transpose(posm)
    posm_out = jnp.transpose(pos_t.reshape(_POS_D, b, k), (1, 2, 0))
    return (
        colm.reshape(b, k, _COLOR_D),
        posm_out,
        idx_flat.reshape(b, k),
        ent[0, 0],
    )
